# Initial kernel scaffold; baseline (speedup 1.0000x reference)
import jax
import jax.numpy as jnp
from jax.experimental import pallas as pl


def kernel(xyz1, xyz2):
    raise NotImplementedError("write your pallas kernel here")


# speedup vs baseline: 1.4558x; 1.4558x over previous
import jax
import jax.numpy as jnp
from jax.experimental import pallas as pl
from jax.experimental.pallas import tpu as pltpu

_B, _P, _Q, _D = 8, 2048, 2048, 3
_QT = 512
_NQ = _Q // _QT

_BETA = 1.0
_GAMMA_EFF = 1.0


def _chamfer_body(x1_ref, x2_ref, part_ref, idx12_ref, idx21_ref):
    x1 = x1_ref[0]
    s1 = jnp.sum(x1 * x1, axis=1, keepdims=True)

    init = (
        jnp.full((_P, 1), jnp.inf, jnp.float32),
        jnp.zeros((_P, 1), jnp.int32),
        jnp.zeros((1, 1), jnp.float32),
    )

    def body(t, carry):
        min12, idx12, sum21 = carry
        x2t = x2_ref[0, pl.ds(t * _QT, _QT), :]
        inner = jax.lax.dot_general(
            x1, x2t, (((1,), (1,)), ((), ())),
            precision=jax.lax.Precision.DEFAULT,
            preferred_element_type=jnp.float32)
        s2t = jnp.sum(x2t * x2t, axis=1, keepdims=True).reshape(1, _QT)
        d = (s1 + s2t) - 2.0 * inner

        m12 = jnp.min(d, axis=1, keepdims=True)
        qi = jax.lax.broadcasted_iota(jnp.int32, (_P, _QT), 1) + t * _QT
        i12 = jnp.min(jnp.where(d == m12, qi, _Q), axis=1, keepdims=True)
        take = m12 < min12
        min12 = jnp.where(take, m12, min12)
        idx12 = jnp.where(take, i12, idx12)

        m21 = jnp.min(d, axis=0, keepdims=True)
        pi = jax.lax.broadcasted_iota(jnp.int32, (_P, _QT), 0)
        i21 = jnp.min(jnp.where(d == m21, pi, _P), axis=0, keepdims=True)
        idx21_ref[0, :, pl.ds(t * _QT, _QT)] = i21
        sum21 = sum21 + jnp.sum(m21).reshape(1, 1)
        return min12, idx12, sum21

    min12, idx12, sum21 = jax.lax.fori_loop(0, _NQ, body, init)
    idx12_ref[0] = idx12
    part = (jnp.sum(min12) / _P
            + _BETA * jnp.max(min12)
            + _GAMMA_EFF * sum21[0, 0] / _Q)
    part_ref[0] = part.reshape(1, 1)


def kernel(xyz1, xyz2):
    part, idx12, idx21 = pl.pallas_call(
        _chamfer_body,
        grid=(_B,),
        in_specs=[
            pl.BlockSpec((1, _P, _D), lambda b: (b, 0, 0)),
            pl.BlockSpec((1, _Q, _D), lambda b: (b, 0, 0)),
        ],
        out_specs=[
            pl.BlockSpec((1, 1, 1), lambda b: (b, 0, 0)),
            pl.BlockSpec((1, _P, 1), lambda b: (b, 0, 0)),
            pl.BlockSpec((1, 1, _Q), lambda b: (b, 0, 0)),
        ],
        out_shape=[
            jax.ShapeDtypeStruct((_B, 1, 1), jnp.float32),
            jax.ShapeDtypeStruct((_B, _P, 1), jnp.int32),
            jax.ShapeDtypeStruct((_B, 1, _Q), jnp.int32),
        ],
        compiler_params=pltpu.CompilerParams(
            dimension_semantics=("parallel",)),
    )(xyz1, xyz2)
    loss = jnp.mean(part)
    return loss, idx12.reshape(_B, _P), idx21.reshape(_B, _Q)


# speedup vs baseline: 1.6900x; 1.1609x over previous
import jax
import jax.numpy as jnp
from jax.experimental import pallas as pl
from jax.experimental.pallas import tpu as pltpu

_B, _P, _Q, _D = 8, 2048, 2048, 3
_QT = 1024
_NQ = _Q // _QT

_BETA = 1.0
_GAMMA_EFF = 1.0


def _chamfer_body(x1_ref, x2_ref, part_ref, idx12_ref, idx21_ref):
    x1 = x1_ref[0]
    s1 = jnp.sum(x1 * x1, axis=1, keepdims=True)
    x1d = x1 + x1

    qi = jax.lax.broadcasted_iota(jnp.int32, (_P, _QT), 1)
    pi = jax.lax.broadcasted_iota(jnp.int32, (_P, _QT), 0)

    init = (
        jnp.full((_P, 1), jnp.inf, jnp.float32),
        jnp.zeros((_P, 1), jnp.int32),
        jnp.zeros((1, 1), jnp.float32),
    )

    def body(t, carry):
        min12, idx12, sum21 = carry
        x2t = x2_ref[0, pl.ds(t * _QT, _QT), :]
        inner2 = jax.lax.dot_general(
            x1d, x2t, (((1,), (1,)), ((), ())),
            precision=jax.lax.Precision.DEFAULT,
            preferred_element_type=jnp.float32)
        s2t = jnp.sum(x2t * x2t, axis=1, keepdims=True).reshape(1, _QT)
        d = (s1 + s2t) - inner2

        m12 = jnp.min(d, axis=1, keepdims=True)
        i12 = jnp.min(jnp.where(d == m12, qi, _Q), axis=1, keepdims=True)
        take = m12 < min12
        min12 = jnp.where(take, m12, min12)
        idx12 = jnp.where(take, i12 + t * _QT, idx12)

        m21 = jnp.min(d, axis=0, keepdims=True)
        i21 = jnp.min(jnp.where(d == m21, pi, _P), axis=0, keepdims=True)
        idx21_ref[0, :, pl.ds(t * _QT, _QT)] = i21
        sum21 = sum21 + jnp.sum(m21).reshape(1, 1)
        return min12, idx12, sum21

    min12, idx12, sum21 = jax.lax.fori_loop(0, _NQ, body, init)
    idx12_ref[0] = idx12
    part = (jnp.sum(min12) / _P
            + _BETA * jnp.max(min12)
            + _GAMMA_EFF * sum21[0, 0] / _Q)
    part_ref[0] = part.reshape(1, 1)


def kernel(xyz1, xyz2):
    part, idx12, idx21 = pl.pallas_call(
        _chamfer_body,
        grid=(_B,),
        in_specs=[
            pl.BlockSpec((1, _P, _D), lambda b: (b, 0, 0)),
            pl.BlockSpec((1, _Q, _D), lambda b: (b, 0, 0)),
        ],
        out_specs=[
            pl.BlockSpec((1, 1, 1), lambda b: (b, 0, 0)),
            pl.BlockSpec((1, _P, 1), lambda b: (b, 0, 0)),
            pl.BlockSpec((1, 1, _Q), lambda b: (b, 0, 0)),
        ],
        out_shape=[
            jax.ShapeDtypeStruct((_B, 1, 1), jnp.float32),
            jax.ShapeDtypeStruct((_B, _P, 1), jnp.int32),
            jax.ShapeDtypeStruct((_B, 1, _Q), jnp.int32),
        ],
        compiler_params=pltpu.CompilerParams(
            dimension_semantics=("parallel",)),
    )(xyz1, xyz2)
    loss = jnp.mean(part)
    return loss, idx12.reshape(_B, _P), idx21.reshape(_B, _Q)


# speedup vs baseline: 2.0193x; 1.1949x over previous
import jax
import jax.numpy as jnp
from jax.experimental import pallas as pl
from jax.experimental.pallas import tpu as pltpu

_B, _P, _Q, _D = 8, 2048, 2048, 3
_QT = 2048
_NQ = _Q // _QT

_BETA = 1.0
_GAMMA_EFF = 1.0


def _chamfer_body(x1_ref, x2_ref, part_ref, idx12_ref, idx21_ref):
    x1 = x1_ref[0]
    s1 = jnp.sum(x1 * x1, axis=1, keepdims=True)
    x1d = x1 + x1

    qi = jax.lax.broadcasted_iota(jnp.int32, (_P, _QT), 1)
    pi = jax.lax.broadcasted_iota(jnp.int32, (_P, _QT), 0)

    init = (
        jnp.full((_P, 1), jnp.inf, jnp.float32),
        jnp.zeros((_P, 1), jnp.int32),
        jnp.zeros((1, 1), jnp.float32),
    )

    def body(t, carry):
        min12, idx12, sum21 = carry
        x2t = x2_ref[0, pl.ds(t * _QT, _QT), :]
        inner2 = jax.lax.dot_general(
            x1d, x2t, (((1,), (1,)), ((), ())),
            precision=jax.lax.Precision.DEFAULT,
            preferred_element_type=jnp.float32)
        s2t = jnp.sum(x2t * x2t, axis=1, keepdims=True).reshape(1, _QT)
        d = (s1 + s2t) - inner2

        m12 = jnp.min(d, axis=1, keepdims=True)
        i12 = jnp.min(jnp.where(d == m12, qi, _Q), axis=1, keepdims=True)
        take = m12 < min12
        min12 = jnp.where(take, m12, min12)
        idx12 = jnp.where(take, i12 + t * _QT, idx12)

        m21 = jnp.min(d, axis=0, keepdims=True)
        i21 = jnp.min(jnp.where(d == m21, pi, _P), axis=0, keepdims=True)
        idx21_ref[0, :, pl.ds(t * _QT, _QT)] = i21
        sum21 = sum21 + jnp.sum(m21).reshape(1, 1)
        return min12, idx12, sum21

    min12, idx12, sum21 = jax.lax.fori_loop(0, _NQ, body, init)
    idx12_ref[0] = idx12
    part = (jnp.sum(min12) / _P
            + _BETA * jnp.max(min12)
            + _GAMMA_EFF * sum21[0, 0] / _Q)
    part_ref[0] = part.reshape(1, 1)


def kernel(xyz1, xyz2):
    part, idx12, idx21 = pl.pallas_call(
        _chamfer_body,
        grid=(_B,),
        in_specs=[
            pl.BlockSpec((1, _P, _D), lambda b: (b, 0, 0)),
            pl.BlockSpec((1, _Q, _D), lambda b: (b, 0, 0)),
        ],
        out_specs=[
            pl.BlockSpec((1, 1, 1), lambda b: (b, 0, 0)),
            pl.BlockSpec((1, _P, 1), lambda b: (b, 0, 0)),
            pl.BlockSpec((1, 1, _Q), lambda b: (b, 0, 0)),
        ],
        out_shape=[
            jax.ShapeDtypeStruct((_B, 1, 1), jnp.float32),
            jax.ShapeDtypeStruct((_B, _P, 1), jnp.int32),
            jax.ShapeDtypeStruct((_B, 1, _Q), jnp.int32),
        ],
        compiler_params=pltpu.CompilerParams(
            dimension_semantics=("parallel",)),
    )(xyz1, xyz2)
    loss = jnp.mean(part)
    return loss, idx12.reshape(_B, _P), idx21.reshape(_B, _Q)


# speedup vs baseline: 2.3803x; 1.1788x over previous
import jax
import jax.numpy as jnp
from jax.experimental import pallas as pl
from jax.experimental.pallas import tpu as pltpu

_B, _P, _Q, _D = 8, 2048, 2048, 3
_QT = 2048
_NQ = _Q // _QT

_BETA = 1.0
_GAMMA_EFF = 1.0


def _argmin_lanes(d):
    rows, cols = d.shape
    w = cols // 2
    mask = d[:, w:] < d[:, :w]
    v = jnp.where(mask, d[:, w:], d[:, :w])
    base = jax.lax.broadcasted_iota(jnp.int32, (rows, w), 1)
    idx = jnp.where(mask, base + w, base)
    w //= 2
    while w >= 128:
        mask = v[:, w:] < v[:, :w]
        v = jnp.where(mask, v[:, w:], v[:, :w])
        idx = jnp.where(mask, idx[:, w:], idx[:, :w])
        w //= 2
    m = jnp.min(v, axis=1, keepdims=True)
    i = jnp.min(jnp.where(v == m, idx, cols), axis=1, keepdims=True)
    return m, i


def _argmin_sublanes(d):
    rows, cols = d.shape
    h = rows // 2
    mask = d[h:, :] < d[:h, :]
    v = jnp.where(mask, d[h:, :], d[:h, :])
    base = jax.lax.broadcasted_iota(jnp.int32, (h, cols), 0)
    idx = jnp.where(mask, base + h, base)
    h //= 2
    while h >= 8:
        mask = v[h:, :] < v[:h, :]
        v = jnp.where(mask, v[h:, :], v[:h, :])
        idx = jnp.where(mask, idx[h:, :], idx[:h, :])
        h //= 2
    m = jnp.min(v, axis=0, keepdims=True)
    i = jnp.min(jnp.where(v == m, idx, rows), axis=0, keepdims=True)
    return m, i


def _chamfer_body(x1_ref, x2_ref, part_ref, idx12_ref, idx21_ref):
    x1 = x1_ref[0]
    s1 = jnp.sum(x1 * x1, axis=1, keepdims=True)
    x1d = x1 + x1
    x2 = x2_ref[0]

    inner2 = jax.lax.dot_general(
        x1d, x2, (((1,), (1,)), ((), ())),
        precision=jax.lax.Precision.DEFAULT,
        preferred_element_type=jnp.float32)
    s2 = jnp.sum(x2 * x2, axis=1, keepdims=True).reshape(1, _Q)
    d = (s1 + s2) - inner2

    min12, idx12 = _argmin_lanes(d)
    m21, i21 = _argmin_sublanes(d)
    idx21_ref[0] = i21
    sum21 = jnp.sum(m21)
    idx12_ref[0] = idx12
    part = (jnp.sum(min12) / _P
            + _BETA * jnp.max(min12)
            + _GAMMA_EFF * sum21 / _Q)
    part_ref[0] = part.reshape(1, 1)


def kernel(xyz1, xyz2):
    part, idx12, idx21 = pl.pallas_call(
        _chamfer_body,
        grid=(_B,),
        in_specs=[
            pl.BlockSpec((1, _P, _D), lambda b: (b, 0, 0)),
            pl.BlockSpec((1, _Q, _D), lambda b: (b, 0, 0)),
        ],
        out_specs=[
            pl.BlockSpec((1, 1, 1), lambda b: (b, 0, 0)),
            pl.BlockSpec((1, _P, 1), lambda b: (b, 0, 0)),
            pl.BlockSpec((1, 1, _Q), lambda b: (b, 0, 0)),
        ],
        out_shape=[
            jax.ShapeDtypeStruct((_B, 1, 1), jnp.float32),
            jax.ShapeDtypeStruct((_B, _P, 1), jnp.int32),
            jax.ShapeDtypeStruct((_B, 1, _Q), jnp.int32),
        ],
        compiler_params=pltpu.CompilerParams(
            dimension_semantics=("parallel",)),
    )(xyz1, xyz2)
    loss = jnp.mean(part)
    return loss, idx12.reshape(_B, _P), idx21.reshape(_B, _Q)
